# baseline (speedup 1.0000x reference)
import functools
import numpy as np
import jax
import jax.numpy as jnp
from jax import lax
from jax.experimental import pallas as pl
from jax.experimental.pallas import tpu as pltpu
from jax.experimental.pallas import tpu_sc as plsc

DIM = 256
HEADS = 8
HD = DIM // HEADS
WIN = 8
E = 8
SCALE = HD ** -0.5
RHID = 128
BLK = 512
T_TOK = 8192
NBLK = T_TOK // BLK
TILE = 512
NT = T_TOK // TILE + E - 1
NSLOT = NT * TILE
NC_SC = 2
NS_SC = 16
NW_SC = NC_SC * NS_SC
PER_W = T_TOK // NW_SC


def _rope_tables():
    d = HD // 2
    half = d // 2
    inv = 1.0 / (10000.0 ** (np.arange(half, dtype=np.float64) / half))
    hpos = np.repeat(np.arange(WIN), WIN).astype(np.float64)
    wpos = np.tile(np.arange(WIN), WIN).astype(np.float64)
    ah = hpos[:, None] * inv[None, :]
    aw = wpos[:, None] * inv[None, :]
    cos = np.concatenate([np.cos(ah), np.cos(ah), np.cos(aw), np.cos(aw)], axis=-1)
    sin = np.concatenate([np.sin(ah), np.sin(ah), np.sin(aw), np.sin(aw)], axis=-1)
    t = np.arange(BLK)
    p = (t // 64) * WIN + (t % WIN)
    cos_b = np.tile(cos[p], (1, 2 * HEADS)).astype(np.float32)
    sin_b = np.tile(sin[p], (1, 2 * HEADS)).astype(np.float32)
    return cos_b, sin_b


_COS_B, _SIN_B = _rope_tables()
_TRIL = np.tril(np.ones((BLK, BLK), np.float32), -1)
_TRIU = np.triu(np.ones((E, E), np.float32), 1)
_GRP = 256
_AMASK = np.where(
    (np.arange(_GRP)[:, None] // 64) == (np.arange(_GRP)[None, :] // 64),
    0.0, -1e30).astype(np.float32)


def _rot_half_qk(x):
    pieces = []
    for g in range(2 * HEADS):
        b = g * HD
        pieces += [-x[:, b + 8:b + 16], x[:, b:b + 8],
                   -x[:, b + 24:b + 32], x[:, b + 16:b + 24]]
    return jnp.concatenate(pieces, axis=1)


def _rd_body(x_ref, wr1_ref, br1_ref, wr2_ref, br2_ref, tril_ref, triu_ref,
             ohp_ref, dst_ref, te_ref, ohs_s, tot_s, base_s):
    i = pl.program_id(0)

    @pl.when(i == 0)
    def _init0():
        tot_s[...] = jnp.zeros((1, E), jnp.float32)

    @pl.when(i < NBLK)
    def _phase_router():
        x = x_ref[...]
        hid = jnp.maximum(
            jnp.dot(x, wr1_ref[...], preferred_element_type=jnp.float32)
            + br1_ref[...], 0.0)
        logits = (jnp.dot(hid, wr2_ref[...], preferred_element_type=jnp.float32)
                  + br2_ref[...])
        mx = jnp.max(logits, axis=-1, keepdims=True)
        ex = jnp.exp(logits - mx)
        probs = ex / jnp.sum(ex, axis=-1, keepdims=True)
        pmax = jnp.max(probs, axis=-1, keepdims=True)
        masks = []
        found = jnp.zeros((BLK, 1), jnp.float32)
        for e in range(E):
            col = probs[:, e:e + 1]
            is_max = jnp.where(col >= pmax, 1.0, 0.0) * (1.0 - found)
            masks.append(is_max)
            found = found + is_max
        oh = jnp.concatenate(masks, axis=1)
        ohp = oh * pmax
        ohp_ref[...] = ohp
        ohs_s[i] = ohp
        tot_s[...] = tot_s[...] + jnp.sum(oh, axis=0, keepdims=True)

    @pl.when(i == NBLK)
    def _init1():
        base_s[...] = jnp.zeros((1, E), jnp.float32)

    @pl.when(i >= NBLK)
    def _phase_dispatch():
        blk = i - NBLK
        tot_i = tot_s[...].astype(jnp.int32)
        tiles = jnp.right_shift(tot_i + (TILE - 1), 9)
        pad_off = jnp.dot(tiles.astype(jnp.float32), triu_ref[...],
                          preferred_element_type=jnp.float32) * TILE

        oh = jnp.where(ohs_s[blk] > 0.0, 1.0, 0.0)
        prefix = jnp.dot(tril_ref[...], oh,
                         preferred_element_type=jnp.float32)
        offs = pad_off + base_s[...]
        dstf = jnp.sum(oh * (prefix + offs), axis=1, keepdims=True)
        dst_ref[...] = jnp.broadcast_to(dstf.astype(jnp.int32), (BLK, E))
        base_s[...] = base_s[...] + jnp.sum(oh, axis=0, keepdims=True)

        trow = lax.broadcasted_iota(jnp.int32, (32, E), 0) * TILE
        cnt = jnp.sum(jnp.where(trow >= pad_off.astype(jnp.int32), 1, 0),
                      axis=1, keepdims=True) - 1
        te_ref[...] = jnp.broadcast_to(cnt, (32, E))


def _route_dispatch(xf, Wr1, br1, Wr2, br2):
    return pl.pallas_call(
        _rd_body,
        grid=(2 * NBLK,),
        in_specs=[
            pl.BlockSpec((BLK, DIM), lambda i: (jnp.minimum(i, NBLK - 1), 0)),
            pl.BlockSpec((DIM, RHID), lambda i: (0, 0)),
            pl.BlockSpec((1, RHID), lambda i: (0, 0)),
            pl.BlockSpec((RHID, E), lambda i: (0, 0)),
            pl.BlockSpec((1, E), lambda i: (0, 0)),
            pl.BlockSpec((BLK, BLK), lambda i: (0, 0)),
            pl.BlockSpec((E, E), lambda i: (0, 0)),
        ],
        out_specs=[
            pl.BlockSpec((BLK, E), lambda i: (jnp.minimum(i, NBLK - 1), 0)),
            pl.BlockSpec((BLK, E), lambda i: (jnp.maximum(i - NBLK, 0), 0)),
            pl.BlockSpec((32, E), lambda i: (0, 0)),
        ],
        out_shape=[
            jax.ShapeDtypeStruct((T_TOK, E), jnp.float32),
            jax.ShapeDtypeStruct((T_TOK, E), jnp.int32),
            jax.ShapeDtypeStruct((32, E), jnp.int32),
        ],
        scratch_shapes=[
            pltpu.VMEM((NBLK, BLK, E), jnp.float32),
            pltpu.VMEM((1, E), jnp.float32),
            pltpu.VMEM((1, E), jnp.float32),
        ],
    )(xf, Wr1, br1.reshape(1, RHID), Wr2, br2.reshape(1, E),
      jnp.asarray(_TRIL), jnp.asarray(_TRIU))


def _sc_permute(direction, width, chunk):
    mesh = plsc.VectorSubcoreMesh(core_axis_name="c", subcore_axis_name="s")
    nrows_out = NSLOT if direction == "scatter" else T_TOK
    nch = PER_W // chunk

    @functools.partial(
        pl.kernel, mesh=mesh,
        out_type=jax.ShapeDtypeStruct((nrows_out, width), jnp.float32),
        scratch_types=[
            pltpu.VMEM((chunk,), jnp.int32),
            pltpu.VMEM((chunk, width), jnp.float32),
            pltpu.SemaphoreType.DMA,
        ])
    def k(src_hbm, dst_hbm, out_hbm, idx_v, buf_v, sem):
        wid = lax.axis_index("s") * NC_SC + lax.axis_index("c")
        base = wid * PER_W
        for ch in range(nch):
            b = base + ch * chunk
            pltpu.sync_copy(dst_hbm.at[pl.ds(b, chunk)], idx_v)
            if direction == "scatter":
                pltpu.sync_copy(src_hbm.at[pl.ds(b, chunk)], buf_v)
                pltpu.async_copy(buf_v, out_hbm.at[idx_v], sem).wait()
            else:
                pltpu.async_copy(src_hbm.at[idx_v], buf_v, sem).wait()
                pltpu.sync_copy(buf_v, out_hbm.at[pl.ds(b, chunk)])

    return k


def _sc_scatter_rows(src, dst1, width, chunk):
    return _sc_permute("scatter", width, chunk)(src, dst1)


def _sc_gather_rows(src, dst1, width, chunk):
    return _sc_permute("gather", width, chunk)(src, dst1)


def _grouped_matmul(xs, te, W, b):
    width = W.shape[-1]

    def body(te_ref, xs_ref, w_ref, b_ref, o_ref):
        o_ref[...] = (jnp.dot(xs_ref[...].astype(jnp.bfloat16), w_ref[0],
                              preferred_element_type=jnp.float32)
                      + b_ref[0])

    grid_spec = pltpu.PrefetchScalarGridSpec(
        num_scalar_prefetch=1,
        grid=(NT,),
        in_specs=[
            pl.BlockSpec((TILE, DIM), lambda t, te: (t, 0)),
            pl.BlockSpec((1, DIM, width), lambda t, te: (te[t], 0, 0)),
            pl.BlockSpec((1, 1, width), lambda t, te: (te[t], 0, 0)),
        ],
        out_specs=pl.BlockSpec((TILE, width), lambda t, te: (t, 0)),
    )
    return pl.pallas_call(
        body, grid_spec=grid_spec,
        out_shape=jax.ShapeDtypeStruct((NSLOT, width), jnp.float32),
    )(te, xs, W, b.reshape(E, 1, width))


def _attn_body(qkv_ref, ohp_ref, cos_ref, sin_ref, mask_ref, wp_ref, bp_ref,
               out_ref, qkv_s, win2_s, att_s):
    qkv = qkv_ref[...]
    qk = qkv[:, :2 * DIM]
    qk = qk * cos_ref[...] + _rot_half_qk(qk) * sin_ref[...]
    qkv_s[...] = jnp.concatenate([qk, qkv[:, 2 * DIM:]], axis=1
                                 ).reshape(WIN, 64, 3 * DIM)

    for w in range(WIN):
        win2_s[w] = qkv_s[:, w * WIN:(w + 1) * WIN, :].reshape(
            WIN * WIN, 3 * DIM)
    big = win2_s[...].reshape(BLK, 3 * DIM)
    mask = mask_ref[...]
    grps = []
    for g in range(BLK // _GRP):
        rows = slice(g * _GRP, (g + 1) * _GRP)
        outs = []
        for h in range(HEADS):
            qh = (big[rows, h * HD:(h + 1) * HD] * SCALE
                  ).astype(jnp.bfloat16)
            kh = big[rows, DIM + h * HD:DIM + (h + 1) * HD
                     ].astype(jnp.bfloat16)
            vh = big[rows, 2 * DIM + h * HD:2 * DIM + (h + 1) * HD
                     ].astype(jnp.bfloat16)
            s = lax.dot_general(qh, kh, (((1,), (1,)), ((), ())),
                                preferred_element_type=jnp.float32) + mask
            p = jnp.exp(s - jnp.max(s, axis=-1, keepdims=True))
            r = 1.0 / jnp.sum(p, axis=-1, keepdims=True)
            o = jnp.dot(p.astype(jnp.bfloat16), vh,
                        preferred_element_type=jnp.float32)
            outs.append(o * r)
        grps.append(jnp.concatenate(outs, axis=1))
    big_out = jnp.concatenate(grps, axis=0)
    for w in range(WIN):
        att_s[:, w * WIN:(w + 1) * WIN, :] = big_out[
            w * WIN * WIN:(w + 1) * WIN * WIN, :].reshape(WIN, WIN, DIM)

    ohp = ohp_ref[...]
    pmax = jnp.sum(ohp, axis=1, keepdims=True)
    o = att_s[...].reshape(BLK, DIM).astype(jnp.bfloat16)
    bp = bp_ref[...]
    acc = jnp.zeros((BLK, DIM), jnp.float32)
    for e in range(E):
        emask = jnp.where(ohp[:, e:e + 1] > 0.0, 1.0, 0.0)
        acc = acc + emask * (
            jnp.dot(o, wp_ref[e], preferred_element_type=jnp.float32)
            + bp[e:e + 1, :])
    out_ref[...] = acc * pmax


def _attention(qkv, ohp, Wproj, bproj):
    return pl.pallas_call(
        _attn_body,
        grid=(NBLK,),
        in_specs=[
            pl.BlockSpec((BLK, 3 * DIM), lambda i: (i, 0)),
            pl.BlockSpec((BLK, E), lambda i: (i, 0)),
            pl.BlockSpec((BLK, 2 * DIM), lambda i: (0, 0)),
            pl.BlockSpec((BLK, 2 * DIM), lambda i: (0, 0)),
            pl.BlockSpec((_GRP, _GRP), lambda i: (0, 0)),
            pl.BlockSpec((E, DIM, DIM), lambda i: (0, 0, 0)),
            pl.BlockSpec((E, DIM), lambda i: (0, 0)),
        ],
        out_specs=pl.BlockSpec((BLK, DIM), lambda i: (i, 0)),
        out_shape=jax.ShapeDtypeStruct((T_TOK, DIM), jnp.float32),
        scratch_shapes=[
            pltpu.VMEM((WIN, 64, 3 * DIM), jnp.float32),
            pltpu.VMEM((WIN, WIN * WIN, 3 * DIM), jnp.float32),
            pltpu.VMEM((WIN, 64, DIM), jnp.float32),
        ],
    )(qkv, ohp, jnp.asarray(_COS_B), jnp.asarray(_SIN_B),
      jnp.asarray(_AMASK), Wproj, bproj)


@jax.jit
def kernel(x, Wqkv, bqkv, Wproj, bproj, Wr1, br1, Wr2, br2):
    Bs, H, W, C = x.shape
    xf = x.reshape(-1, C)
    ohp, dstrep, te_out = _route_dispatch(xf, Wr1, br1, Wr2, br2)
    te = te_out[:NT, 0]
    dst1 = dstrep[:, 0]
    xs = _sc_scatter_rows(xf, dst1, DIM, 128)
    qkv_s = _grouped_matmul(xs, te, Wqkv.astype(jnp.bfloat16), bqkv)
    qkv = _sc_gather_rows(qkv_s, dst1, 3 * DIM, 64)
    final = _attention(qkv, ohp, Wproj.astype(jnp.bfloat16), bproj)
    return final.reshape(Bs, H, W, C)


# speedup vs baseline: 1.0523x; 1.0523x over previous
import functools
import numpy as np
import jax
import jax.numpy as jnp
from jax import lax
from jax.experimental import pallas as pl
from jax.experimental.pallas import tpu as pltpu
from jax.experimental.pallas import tpu_sc as plsc

DIM = 256
HEADS = 8
HD = DIM // HEADS
WIN = 8
E = 8
SCALE = HD ** -0.5
RHID = 128
BLK = 512
T_TOK = 8192
NBLK = T_TOK // BLK
TILE = 512
NT = T_TOK // TILE + E - 1
NSLOT = NT * TILE
NC_SC = 2
NS_SC = 16
NW_SC = NC_SC * NS_SC
PER_W = T_TOK // NW_SC


def _rope_tables():
    d = HD // 2
    half = d // 2
    inv = 1.0 / (10000.0 ** (np.arange(half, dtype=np.float64) / half))
    hpos = np.repeat(np.arange(WIN), WIN).astype(np.float64)
    wpos = np.tile(np.arange(WIN), WIN).astype(np.float64)
    ah = hpos[:, None] * inv[None, :]
    aw = wpos[:, None] * inv[None, :]
    cos = np.concatenate([np.cos(ah), np.cos(ah), np.cos(aw), np.cos(aw)], axis=-1)
    sin = np.concatenate([np.sin(ah), np.sin(ah), np.sin(aw), np.sin(aw)], axis=-1)
    t = np.arange(BLK)
    p = (t // 64) * WIN + (t % WIN)
    cos_b = np.tile(cos[p], (1, 2 * HEADS)).astype(np.float32)
    sin_b = np.tile(sin[p], (1, 2 * HEADS)).astype(np.float32)
    return cos_b, sin_b


_COS_B, _SIN_B = _rope_tables()
_TRIL = np.tril(np.ones((BLK, BLK), np.float32), -1)
_TRIU = np.triu(np.ones((E, E), np.float32), 1)
_GRP = 512
_AMASK = np.where(
    (np.arange(_GRP)[:, None] // 64) == (np.arange(_GRP)[None, :] // 64),
    0.0, -1e30).astype(np.float32)


def _rot_half_qk(x):
    pieces = []
    for g in range(2 * HEADS):
        b = g * HD
        pieces += [-x[:, b + 8:b + 16], x[:, b:b + 8],
                   -x[:, b + 24:b + 32], x[:, b + 16:b + 24]]
    return jnp.concatenate(pieces, axis=1)


def _rd_body(x_ref, wr1_ref, br1_ref, wr2_ref, br2_ref, tril_ref, triu_ref,
             ohp_ref, dst_ref, te_ref, ohs_s, tot_s, base_s):
    i = pl.program_id(0)

    @pl.when(i == 0)
    def _init0():
        tot_s[...] = jnp.zeros((1, E), jnp.float32)

    @pl.when(i < NBLK)
    def _phase_router():
        x = x_ref[...]
        hid = jnp.maximum(
            jnp.dot(x, wr1_ref[...], preferred_element_type=jnp.float32)
            + br1_ref[...], 0.0)
        logits = (jnp.dot(hid, wr2_ref[...], preferred_element_type=jnp.float32)
                  + br2_ref[...])
        mx = jnp.max(logits, axis=-1, keepdims=True)
        ex = jnp.exp(logits - mx)
        probs = ex / jnp.sum(ex, axis=-1, keepdims=True)
        pmax = jnp.max(probs, axis=-1, keepdims=True)
        masks = []
        found = jnp.zeros((BLK, 1), jnp.float32)
        for e in range(E):
            col = probs[:, e:e + 1]
            is_max = jnp.where(col >= pmax, 1.0, 0.0) * (1.0 - found)
            masks.append(is_max)
            found = found + is_max
        oh = jnp.concatenate(masks, axis=1)
        ohp = oh * pmax
        ohp_ref[...] = ohp
        ohs_s[i] = ohp
        tot_s[...] = tot_s[...] + jnp.sum(oh, axis=0, keepdims=True)

    @pl.when(i == NBLK)
    def _init1():
        base_s[...] = jnp.zeros((1, E), jnp.float32)

    @pl.when(i >= NBLK)
    def _phase_dispatch():
        blk = i - NBLK
        tot_i = tot_s[...].astype(jnp.int32)
        tiles = jnp.right_shift(tot_i + (TILE - 1), 9)
        pad_off = jnp.dot(tiles.astype(jnp.float32), triu_ref[...],
                          preferred_element_type=jnp.float32) * TILE

        oh = jnp.where(ohs_s[blk] > 0.0, 1.0, 0.0)
        prefix = jnp.dot(tril_ref[...], oh,
                         preferred_element_type=jnp.float32)
        offs = pad_off + base_s[...]
        dstf = jnp.sum(oh * (prefix + offs), axis=1, keepdims=True)
        dst_ref[...] = jnp.broadcast_to(dstf.astype(jnp.int32), (BLK, E))
        base_s[...] = base_s[...] + jnp.sum(oh, axis=0, keepdims=True)

        trow = lax.broadcasted_iota(jnp.int32, (32, E), 0) * TILE
        cnt = jnp.sum(jnp.where(trow >= pad_off.astype(jnp.int32), 1, 0),
                      axis=1, keepdims=True) - 1
        te_ref[...] = jnp.broadcast_to(cnt, (32, E))


def _route_dispatch(xf, Wr1, br1, Wr2, br2):
    return pl.pallas_call(
        _rd_body,
        grid=(2 * NBLK,),
        in_specs=[
            pl.BlockSpec((BLK, DIM), lambda i: (jnp.minimum(i, NBLK - 1), 0)),
            pl.BlockSpec((DIM, RHID), lambda i: (0, 0)),
            pl.BlockSpec((1, RHID), lambda i: (0, 0)),
            pl.BlockSpec((RHID, E), lambda i: (0, 0)),
            pl.BlockSpec((1, E), lambda i: (0, 0)),
            pl.BlockSpec((BLK, BLK), lambda i: (0, 0)),
            pl.BlockSpec((E, E), lambda i: (0, 0)),
        ],
        out_specs=[
            pl.BlockSpec((BLK, E), lambda i: (jnp.minimum(i, NBLK - 1), 0)),
            pl.BlockSpec((BLK, E), lambda i: (jnp.maximum(i - NBLK, 0), 0)),
            pl.BlockSpec((32, E), lambda i: (0, 0)),
        ],
        out_shape=[
            jax.ShapeDtypeStruct((T_TOK, E), jnp.float32),
            jax.ShapeDtypeStruct((T_TOK, E), jnp.int32),
            jax.ShapeDtypeStruct((32, E), jnp.int32),
        ],
        scratch_shapes=[
            pltpu.VMEM((NBLK, BLK, E), jnp.float32),
            pltpu.VMEM((1, E), jnp.float32),
            pltpu.VMEM((1, E), jnp.float32),
        ],
    )(xf, Wr1, br1.reshape(1, RHID), Wr2, br2.reshape(1, E),
      jnp.asarray(_TRIL), jnp.asarray(_TRIU))


def _sc_permute(direction, width, chunk):
    mesh = plsc.VectorSubcoreMesh(core_axis_name="c", subcore_axis_name="s")
    nrows_out = NSLOT if direction == "scatter" else T_TOK
    nch = PER_W // chunk

    @functools.partial(
        pl.kernel, mesh=mesh,
        out_type=jax.ShapeDtypeStruct((nrows_out, width), jnp.float32),
        scratch_types=[
            pltpu.VMEM((chunk,), jnp.int32),
            pltpu.VMEM((chunk, width), jnp.float32),
            pltpu.SemaphoreType.DMA,
        ])
    def k(src_hbm, dst_hbm, out_hbm, idx_v, buf_v, sem):
        wid = lax.axis_index("s") * NC_SC + lax.axis_index("c")
        base = wid * PER_W
        for ch in range(nch):
            b = base + ch * chunk
            pltpu.sync_copy(dst_hbm.at[pl.ds(b, chunk)], idx_v)
            if direction == "scatter":
                pltpu.sync_copy(src_hbm.at[pl.ds(b, chunk)], buf_v)
                pltpu.async_copy(buf_v, out_hbm.at[idx_v], sem).wait()
            else:
                pltpu.async_copy(src_hbm.at[idx_v], buf_v, sem).wait()
                pltpu.sync_copy(buf_v, out_hbm.at[pl.ds(b, chunk)])

    return k


def _sc_scatter_rows(src, dst1, width, chunk):
    return _sc_permute("scatter", width, chunk)(src, dst1)


def _sc_gather_rows(src, dst1, width, chunk):
    return _sc_permute("gather", width, chunk)(src, dst1)


def _grouped_matmul(xs, te, W, b):
    width = W.shape[-1]

    def body(te_ref, xs_ref, w_ref, b_ref, o_ref):
        o_ref[...] = (jnp.dot(xs_ref[...].astype(jnp.bfloat16), w_ref[0],
                              preferred_element_type=jnp.float32)
                      + b_ref[0])

    grid_spec = pltpu.PrefetchScalarGridSpec(
        num_scalar_prefetch=1,
        grid=(NT,),
        in_specs=[
            pl.BlockSpec((TILE, DIM), lambda t, te: (t, 0)),
            pl.BlockSpec((1, DIM, width), lambda t, te: (te[t], 0, 0)),
            pl.BlockSpec((1, 1, width), lambda t, te: (te[t], 0, 0)),
        ],
        out_specs=pl.BlockSpec((TILE, width), lambda t, te: (t, 0)),
    )
    return pl.pallas_call(
        body, grid_spec=grid_spec,
        out_shape=jax.ShapeDtypeStruct((NSLOT, width), jnp.float32),
    )(te, xs, W, b.reshape(E, 1, width))


def _attn_body(qkv_ref, ohp_ref, cos_ref, sin_ref, mask_ref, wp_ref, bp_ref,
               out_ref, qkv_s, win2_s, att_s):
    qkv = qkv_ref[...]
    qk = qkv[:, :2 * DIM]
    qk = qk * cos_ref[...] + _rot_half_qk(qk) * sin_ref[...]
    qkv_s[...] = jnp.concatenate([qk, qkv[:, 2 * DIM:]], axis=1
                                 ).reshape(WIN, 64, 3 * DIM)

    for w in range(WIN):
        win2_s[w] = qkv_s[:, w * WIN:(w + 1) * WIN, :].reshape(
            WIN * WIN, 3 * DIM)
    big = win2_s[...].reshape(BLK, 3 * DIM)
    mask = mask_ref[...]
    grps = []
    for g in range(BLK // _GRP):
        rows = slice(g * _GRP, (g + 1) * _GRP)
        outs = []
        for h in range(HEADS):
            qh = (big[rows, h * HD:(h + 1) * HD] * SCALE
                  ).astype(jnp.bfloat16)
            kh = big[rows, DIM + h * HD:DIM + (h + 1) * HD
                     ].astype(jnp.bfloat16)
            vh = big[rows, 2 * DIM + h * HD:2 * DIM + (h + 1) * HD
                     ].astype(jnp.bfloat16)
            s = lax.dot_general(qh, kh, (((1,), (1,)), ((), ())),
                                preferred_element_type=jnp.float32) + mask
            p = jnp.exp(s - jnp.max(s, axis=-1, keepdims=True))
            r = 1.0 / jnp.sum(p, axis=-1, keepdims=True)
            o = jnp.dot(p.astype(jnp.bfloat16), vh,
                        preferred_element_type=jnp.float32)
            outs.append(o * r)
        grps.append(jnp.concatenate(outs, axis=1))
    big_out = jnp.concatenate(grps, axis=0)
    for w in range(WIN):
        att_s[:, w * WIN:(w + 1) * WIN, :] = big_out[
            w * WIN * WIN:(w + 1) * WIN * WIN, :].reshape(WIN, WIN, DIM)

    ohp = ohp_ref[...]
    pmax = jnp.sum(ohp, axis=1, keepdims=True)
    o = att_s[...].reshape(BLK, DIM).astype(jnp.bfloat16)
    bp = bp_ref[...]
    acc = jnp.zeros((BLK, DIM), jnp.float32)
    for e in range(E):
        emask = jnp.where(ohp[:, e:e + 1] > 0.0, 1.0, 0.0)
        acc = acc + emask * (
            jnp.dot(o, wp_ref[e], preferred_element_type=jnp.float32)
            + bp[e:e + 1, :])
    out_ref[...] = acc * pmax


def _attention(qkv, ohp, Wproj, bproj):
    return pl.pallas_call(
        _attn_body,
        grid=(NBLK,),
        in_specs=[
            pl.BlockSpec((BLK, 3 * DIM), lambda i: (i, 0)),
            pl.BlockSpec((BLK, E), lambda i: (i, 0)),
            pl.BlockSpec((BLK, 2 * DIM), lambda i: (0, 0)),
            pl.BlockSpec((BLK, 2 * DIM), lambda i: (0, 0)),
            pl.BlockSpec((_GRP, _GRP), lambda i: (0, 0)),
            pl.BlockSpec((E, DIM, DIM), lambda i: (0, 0, 0)),
            pl.BlockSpec((E, DIM), lambda i: (0, 0)),
        ],
        out_specs=pl.BlockSpec((BLK, DIM), lambda i: (i, 0)),
        out_shape=jax.ShapeDtypeStruct((T_TOK, DIM), jnp.float32),
        scratch_shapes=[
            pltpu.VMEM((WIN, 64, 3 * DIM), jnp.float32),
            pltpu.VMEM((WIN, WIN * WIN, 3 * DIM), jnp.float32),
            pltpu.VMEM((WIN, 64, DIM), jnp.float32),
        ],
    )(qkv, ohp, jnp.asarray(_COS_B), jnp.asarray(_SIN_B),
      jnp.asarray(_AMASK), Wproj, bproj)


@jax.jit
def kernel(x, Wqkv, bqkv, Wproj, bproj, Wr1, br1, Wr2, br2):
    Bs, H, W, C = x.shape
    xf = x.reshape(-1, C)
    ohp, dstrep, te_out = _route_dispatch(xf, Wr1, br1, Wr2, br2)
    te = te_out[:NT, 0]
    dst1 = dstrep[:, 0]
    xs = _sc_scatter_rows(xf, dst1, DIM, 128)
    qkv_s = _grouped_matmul(xs, te, Wqkv.astype(jnp.bfloat16), bqkv)
    qkv = _sc_gather_rows(qkv_s, dst1, 3 * DIM, 64)
    final = _attention(qkv, ohp, Wproj.astype(jnp.bfloat16), bproj)
    return final.reshape(Bs, H, W, C)


# speedup vs baseline: 1.2005x; 1.1409x over previous
import functools
import numpy as np
import jax
import jax.numpy as jnp
from jax import lax
from jax.experimental import pallas as pl
from jax.experimental.pallas import tpu as pltpu
from jax.experimental.pallas import tpu_sc as plsc

DIM = 256
HEADS = 8
HD = DIM // HEADS
WIN = 8
E = 8
SCALE = HD ** -0.5
RHID = 128
BLK = 512
T_TOK = 8192
NBLK = T_TOK // BLK
TILE = 512
NT = T_TOK // TILE + E - 1
NSLOT = NT * TILE
NC_SC = 2
NS_SC = 16
NW_SC = NC_SC * NS_SC
PER_W = T_TOK // NW_SC


def _rope_tables():
    d = HD // 2
    half = d // 2
    inv = 1.0 / (10000.0 ** (np.arange(half, dtype=np.float64) / half))
    hpos = np.repeat(np.arange(WIN), WIN).astype(np.float64)
    wpos = np.tile(np.arange(WIN), WIN).astype(np.float64)
    ah = hpos[:, None] * inv[None, :]
    aw = wpos[:, None] * inv[None, :]
    cos = np.concatenate([np.cos(ah), np.cos(ah), np.cos(aw), np.cos(aw)], axis=-1)
    sin = np.concatenate([np.sin(ah), np.sin(ah), np.sin(aw), np.sin(aw)], axis=-1)
    t = np.arange(BLK)
    p = (t // 64) * WIN + (t % WIN)
    cos_b = np.tile(cos[p], (1, 2 * HEADS)).astype(np.float32)
    sin_b = np.tile(sin[p], (1, 2 * HEADS)).astype(np.float32)
    return cos_b, sin_b


_COS_B, _SIN_B = _rope_tables()
_TRIL = np.tril(np.ones((BLK, BLK), np.float32), -1)
_TRIU = np.triu(np.ones((E, E), np.float32), 1)
_GRP = 512
_AMASK = np.where(
    (np.arange(_GRP)[:, None] // 64) == (np.arange(_GRP)[None, :] // 64),
    0.0, -1e30).astype(np.float32)


def _rot_matrix():
    R = np.zeros((2 * DIM, 2 * DIM), np.float32)
    for g in range(2 * HEADS):
        b = g * HD
        for d in range(8):
            R[b + 8 + d, b + d] = -1.0
            R[b + d, b + 8 + d] = 1.0
            R[b + 24 + d, b + 16 + d] = -1.0
            R[b + 16 + d, b + 24 + d] = 1.0
    return R


_ROT = _rot_matrix()


def _rot_half_qk(x):
    pieces = []
    for g in range(2 * HEADS):
        b = g * HD
        pieces += [-x[:, b + 8:b + 16], x[:, b:b + 8],
                   -x[:, b + 24:b + 32], x[:, b + 16:b + 24]]
    return jnp.concatenate(pieces, axis=1)


def _rd_body(x_ref, wr1_ref, br1_ref, wr2_ref, br2_ref, tril_ref, triu_ref,
             ohp_ref, dst_ref, te_ref, ohs_s, tot_s, base_s):
    i = pl.program_id(0)

    @pl.when(i == 0)
    def _init0():
        tot_s[...] = jnp.zeros((1, E), jnp.float32)

    @pl.when(i < NBLK)
    def _phase_router():
        x = x_ref[...]
        hid = jnp.maximum(
            jnp.dot(x, wr1_ref[...], preferred_element_type=jnp.float32)
            + br1_ref[...], 0.0)
        logits = (jnp.dot(hid, wr2_ref[...], preferred_element_type=jnp.float32)
                  + br2_ref[...])
        mx = jnp.max(logits, axis=-1, keepdims=True)
        ex = jnp.exp(logits - mx)
        probs = ex / jnp.sum(ex, axis=-1, keepdims=True)
        pmax = jnp.max(probs, axis=-1, keepdims=True)
        masks = []
        found = jnp.zeros((BLK, 1), jnp.float32)
        for e in range(E):
            col = probs[:, e:e + 1]
            is_max = jnp.where(col >= pmax, 1.0, 0.0) * (1.0 - found)
            masks.append(is_max)
            found = found + is_max
        oh = jnp.concatenate(masks, axis=1)
        ohp = oh * pmax
        ohp_ref[...] = ohp
        ohs_s[i] = ohp
        tot_s[...] = tot_s[...] + jnp.sum(oh, axis=0, keepdims=True)

    @pl.when(i == NBLK)
    def _init1():
        base_s[...] = jnp.zeros((1, E), jnp.float32)

    @pl.when(i >= NBLK)
    def _phase_dispatch():
        blk = i - NBLK
        tot_i = tot_s[...].astype(jnp.int32)
        tiles = jnp.right_shift(tot_i + (TILE - 1), 9)
        pad_off = jnp.dot(tiles.astype(jnp.float32), triu_ref[...],
                          preferred_element_type=jnp.float32) * TILE

        oh = jnp.where(ohs_s[blk] > 0.0, 1.0, 0.0)
        prefix = jnp.dot(tril_ref[...], oh,
                         preferred_element_type=jnp.float32)
        offs = pad_off + base_s[...]
        dstf = jnp.sum(oh * (prefix + offs), axis=1, keepdims=True)
        dst_ref[...] = jnp.broadcast_to(dstf.astype(jnp.int32), (BLK, E))
        base_s[...] = base_s[...] + jnp.sum(oh, axis=0, keepdims=True)

        trow = lax.broadcasted_iota(jnp.int32, (32, E), 0) * TILE
        cnt = jnp.sum(jnp.where(trow >= pad_off.astype(jnp.int32), 1, 0),
                      axis=1, keepdims=True) - 1
        te_ref[...] = jnp.broadcast_to(cnt, (32, E))


def _route_dispatch(xf, Wr1, br1, Wr2, br2):
    return pl.pallas_call(
        _rd_body,
        grid=(2 * NBLK,),
        in_specs=[
            pl.BlockSpec((BLK, DIM), lambda i: (jnp.minimum(i, NBLK - 1), 0)),
            pl.BlockSpec((DIM, RHID), lambda i: (0, 0)),
            pl.BlockSpec((1, RHID), lambda i: (0, 0)),
            pl.BlockSpec((RHID, E), lambda i: (0, 0)),
            pl.BlockSpec((1, E), lambda i: (0, 0)),
            pl.BlockSpec((BLK, BLK), lambda i: (0, 0)),
            pl.BlockSpec((E, E), lambda i: (0, 0)),
        ],
        out_specs=[
            pl.BlockSpec((BLK, E), lambda i: (jnp.minimum(i, NBLK - 1), 0)),
            pl.BlockSpec((BLK, E), lambda i: (jnp.maximum(i - NBLK, 0), 0)),
            pl.BlockSpec((32, E), lambda i: (0, 0)),
        ],
        out_shape=[
            jax.ShapeDtypeStruct((T_TOK, E), jnp.float32),
            jax.ShapeDtypeStruct((T_TOK, E), jnp.int32),
            jax.ShapeDtypeStruct((32, E), jnp.int32),
        ],
        scratch_shapes=[
            pltpu.VMEM((NBLK, BLK, E), jnp.float32),
            pltpu.VMEM((1, E), jnp.float32),
            pltpu.VMEM((1, E), jnp.float32),
        ],
    )(xf, Wr1, br1.reshape(1, RHID), Wr2, br2.reshape(1, E),
      jnp.asarray(_TRIL), jnp.asarray(_TRIU))


def _sc_permute(direction, width, chunk):
    mesh = plsc.VectorSubcoreMesh(core_axis_name="c", subcore_axis_name="s")
    nrows_out = NSLOT if direction == "scatter" else T_TOK
    nch = PER_W // chunk

    @functools.partial(
        pl.kernel, mesh=mesh,
        out_type=jax.ShapeDtypeStruct((nrows_out, width), jnp.float32),
        scratch_types=[
            pltpu.VMEM((chunk,), jnp.int32),
            pltpu.VMEM((chunk, width), jnp.float32),
            pltpu.SemaphoreType.DMA,
        ])
    def k(src_hbm, dst_hbm, out_hbm, idx_v, buf_v, sem):
        wid = lax.axis_index("s") * NC_SC + lax.axis_index("c")
        base = wid * PER_W
        for ch in range(nch):
            b = base + ch * chunk
            pltpu.sync_copy(dst_hbm.at[pl.ds(b, chunk)], idx_v)
            if direction == "scatter":
                pltpu.sync_copy(src_hbm.at[pl.ds(b, chunk)], buf_v)
                pltpu.async_copy(buf_v, out_hbm.at[idx_v], sem).wait()
            else:
                pltpu.async_copy(src_hbm.at[idx_v], buf_v, sem).wait()
                pltpu.sync_copy(buf_v, out_hbm.at[pl.ds(b, chunk)])

    return k


def _sc_scatter_rows(src, dst1, width, chunk):
    return _sc_permute("scatter", width, chunk)(src, dst1)


def _sc_gather_rows(src, dst1, width, chunk):
    return _sc_permute("gather", width, chunk)(src, dst1)


def _grouped_matmul(xs, te, W, b):
    width = W.shape[-1]

    def body(te_ref, xs_ref, w_ref, b_ref, o_ref):
        o_ref[...] = (jnp.dot(xs_ref[...].astype(jnp.bfloat16), w_ref[0],
                              preferred_element_type=jnp.float32)
                      + b_ref[0])

    grid_spec = pltpu.PrefetchScalarGridSpec(
        num_scalar_prefetch=1,
        grid=(NT,),
        in_specs=[
            pl.BlockSpec((TILE, DIM), lambda t, te: (t, 0)),
            pl.BlockSpec((1, DIM, width), lambda t, te: (te[t], 0, 0)),
            pl.BlockSpec((1, 1, width), lambda t, te: (te[t], 0, 0)),
        ],
        out_specs=pl.BlockSpec((TILE, width), lambda t, te: (t, 0)),
    )
    return pl.pallas_call(
        body, grid_spec=grid_spec,
        out_shape=jax.ShapeDtypeStruct((NSLOT, width), jnp.float32),
    )(te, xs, W, b.reshape(E, 1, width))


def _attn_body(qkv_ref, ohp_ref, cos_ref, sin_ref, mask_ref, rot_ref,
               wp_ref, bp_ref, out_ref, qkv_s, win2_s, att_s):
    qkv = qkv_ref[...]
    qk = qkv[:, :2 * DIM]
    qk = qk * cos_ref[...] + jnp.dot(
        qk, rot_ref[...], preferred_element_type=jnp.float32) * sin_ref[...]
    qkv_s[...] = jnp.concatenate([qk, qkv[:, 2 * DIM:]], axis=1
                                 ).reshape(WIN, 64, 3 * DIM)

    for w in range(WIN):
        win2_s[w] = qkv_s[:, w * WIN:(w + 1) * WIN, :].reshape(
            WIN * WIN, 3 * DIM)
    big = win2_s[...].reshape(BLK, 3 * DIM)
    mask = mask_ref[...]
    grps = []
    for g in range(BLK // _GRP):
        rows = slice(g * _GRP, (g + 1) * _GRP)
        outs = []
        for h in range(HEADS):
            qh = (big[rows, h * HD:(h + 1) * HD] * SCALE
                  ).astype(jnp.bfloat16)
            kh = big[rows, DIM + h * HD:DIM + (h + 1) * HD
                     ].astype(jnp.bfloat16)
            vh = big[rows, 2 * DIM + h * HD:2 * DIM + (h + 1) * HD
                     ].astype(jnp.bfloat16)
            s = lax.dot_general(qh, kh, (((1,), (1,)), ((), ())),
                                preferred_element_type=jnp.float32) + mask
            s = s - jnp.max(s, axis=-1, keepdims=True)
            p = jnp.exp(s)
            p = (p / jnp.sum(p, axis=-1, keepdims=True)).astype(jnp.bfloat16)
            outs.append(jnp.dot(p, vh, preferred_element_type=jnp.float32))
        grps.append(jnp.concatenate(outs, axis=1))
    big_out = jnp.concatenate(grps, axis=0)
    for w in range(WIN):
        att_s[:, w * WIN:(w + 1) * WIN, :] = big_out[
            w * WIN * WIN:(w + 1) * WIN * WIN, :].reshape(WIN, WIN, DIM)

    ohp = ohp_ref[...]
    pmax = jnp.sum(ohp, axis=1, keepdims=True)
    o = att_s[...].reshape(BLK, DIM).astype(jnp.bfloat16)
    bp = bp_ref[...]
    acc = jnp.zeros((BLK, DIM), jnp.float32)
    for e in range(E):
        emask = jnp.where(ohp[:, e:e + 1] > 0.0, 1.0, 0.0)
        acc = acc + emask * (
            jnp.dot(o, wp_ref[e], preferred_element_type=jnp.float32)
            + bp[e:e + 1, :])
    out_ref[...] = acc * pmax


def _attention(qkv, ohp, Wproj, bproj):
    return pl.pallas_call(
        _attn_body,
        grid=(NBLK,),
        in_specs=[
            pl.BlockSpec((BLK, 3 * DIM), lambda i: (i, 0)),
            pl.BlockSpec((BLK, E), lambda i: (i, 0)),
            pl.BlockSpec((BLK, 2 * DIM), lambda i: (0, 0)),
            pl.BlockSpec((BLK, 2 * DIM), lambda i: (0, 0)),
            pl.BlockSpec((_GRP, _GRP), lambda i: (0, 0)),
            pl.BlockSpec((2 * DIM, 2 * DIM), lambda i: (0, 0)),
            pl.BlockSpec((E, DIM, DIM), lambda i: (0, 0, 0)),
            pl.BlockSpec((E, DIM), lambda i: (0, 0)),
        ],
        out_specs=pl.BlockSpec((BLK, DIM), lambda i: (i, 0)),
        out_shape=jax.ShapeDtypeStruct((T_TOK, DIM), jnp.float32),
        scratch_shapes=[
            pltpu.VMEM((WIN, 64, 3 * DIM), jnp.float32),
            pltpu.VMEM((WIN, WIN * WIN, 3 * DIM), jnp.float32),
            pltpu.VMEM((WIN, 64, DIM), jnp.float32),
        ],
    )(qkv, ohp, jnp.asarray(_COS_B), jnp.asarray(_SIN_B),
      jnp.asarray(_AMASK), jnp.asarray(_ROT), Wproj, bproj)


@jax.jit
def kernel(x, Wqkv, bqkv, Wproj, bproj, Wr1, br1, Wr2, br2):
    Bs, H, W, C = x.shape
    xf = x.reshape(-1, C)
    ohp, dstrep, te_out = _route_dispatch(xf, Wr1, br1, Wr2, br2)
    te = te_out[:NT, 0]
    dst1 = dstrep[:, 0]
    xs = _sc_scatter_rows(xf, dst1, DIM, 128)
    qkv_s = _grouped_matmul(xs, te, Wqkv.astype(jnp.bfloat16), bqkv)
    qkv = _sc_gather_rows(qkv_s, dst1, 3 * DIM, 64)
    final = _attention(qkv, ohp, Wproj.astype(jnp.bfloat16), bproj)
    return final.reshape(Bs, H, W, C)
